# Initial kernel scaffold; baseline (speedup 1.0000x reference)
import jax
import jax.numpy as jnp
from jax.experimental import pallas as pl


def kernel(patch_embs, n_patches_list):
    raise NotImplementedError("write your pallas kernel here")


# speedup vs baseline: 1.2919x; 1.2919x over previous
import functools

import jax
import jax.numpy as jnp
from jax import lax
from jax.experimental import pallas as pl
from jax.experimental.pallas import tpu as pltpu
from jax.experimental.pallas import tpu_sc as plsc

ALPHA = 0.2
N_ROWS = 16384
N_COLS = 768
B = 16
NC = 2
NS = 16
NW = NC * NS
ROWS_PER_W = N_ROWS // NW
CHUNK = 64
N_CHUNKS = ROWS_PER_W // CHUNK
LANES = 16
VPR = N_COLS // LANES


def _sc_mix(x, bo_mat, be_mat, bd_mat, lam_vec):
    mesh = plsc.VectorSubcoreMesh(core_axis_name="c", subcore_axis_name="s")

    @functools.partial(
        pl.kernel,
        out_type=jax.ShapeDtypeStruct((N_ROWS, N_COLS), jnp.float32),
        mesh=mesh,
        compiler_params=pltpu.CompilerParams(needs_layout_passes=False),
        scratch_types=[
            pltpu.VMEM((B, LANES), jnp.int32),
            pltpu.VMEM((B, LANES), jnp.int32),
            pltpu.VMEM((B, LANES), jnp.int32),
            pltpu.VMEM((LANES,), jnp.float32),
            pltpu.VMEM((CHUNK, N_COLS), jnp.float32),
            pltpu.VMEM((CHUNK, N_COLS), jnp.float32),
            pltpu.SemaphoreType.DMA,
            pltpu.SemaphoreType.DMA,
        ],
    )
    def kfn(x_hbm, bo_hbm, be_hbm, bd_hbm, lam_hbm, out_hbm,
            bo_v, be_v, bd_v, lam_v, xbuf, obuf, sem_a, sem_b):
        cid = lax.axis_index("c")
        sid = lax.axis_index("s")
        wid = sid * NC + cid

        pltpu.sync_copy(bo_hbm, bo_v)
        pltpu.sync_copy(be_hbm, be_v)
        pltpu.sync_copy(bd_hbm, bd_v)
        pltpu.sync_copy(lam_hbm, lam_v)

        bo = [bo_v[k, :] for k in range(B)]
        be = [be_v[k, :] for k in range(B)]
        bd = [bd_v[k, :] for k in range(B)]
        lam_r = lam_v[...]

        base0 = wid * ROWS_PER_W
        for c in range(N_CHUNKS):
            base = base0 + c * CHUNK
            cp1 = pltpu.async_copy(x_hbm.at[pl.ds(base, CHUNK)], xbuf, sem_a)
            cps = []
            for v in range(CHUNK // LANES):
                rv = base + v * LANES + lax.iota(jnp.int32, LANES)
                src = rv
                for k in range(B):
                    msk = (rv >= bo[k]) & (rv < be[k])
                    src = jnp.where(msk, rv + bd[k], src)
                cps.append(pltpu.async_copy(
                    x_hbm.at[src], obuf.at[pl.ds(v * LANES, LANES)], sem_b))
            cp1.wait()
            for cp in cps:
                cp.wait()

            def row_body(r, carry):
                for d in range(VPR):
                    sl = pl.ds(d * LANES, LANES)
                    xs = xbuf[r, sl]
                    ot = obuf[r, sl]
                    xbuf[r, sl] = ot + lam_r * (xs - ot)
                return carry

            lax.fori_loop(0, CHUNK, row_body, 0)
            pltpu.sync_copy(xbuf, out_hbm.at[pl.ds(base, CHUNK)])

    return kfn(x, bo_mat, be_mat, bd_mat, lam_vec)


def kernel(patch_embs, n_patches_list):
    key = jax.random.key(42)
    ka, kb = jax.random.split(key)
    lam = jax.random.beta(ka, ALPHA, ALPHA)
    lam = jnp.maximum(lam, 1.0 - lam)
    perm = jax.random.permutation(kb, B).astype(jnp.int32)

    sizes = n_patches_list.astype(jnp.int32)
    offs = jnp.concatenate(
        [jnp.zeros((1,), jnp.int32), jnp.cumsum(sizes)[:-1]])
    n_mix = jnp.minimum(sizes, sizes[perm])
    ends = offs + n_mix
    dlt = offs[perm] - offs
    bo_mat = jnp.broadcast_to(offs[:, None], (B, LANES))
    be_mat = jnp.broadcast_to(ends[:, None], (B, LANES))
    bd_mat = jnp.broadcast_to(dlt[:, None], (B, LANES))
    lam_vec = jnp.full((LANES,), lam, dtype=jnp.float32)

    mixed = _sc_mix(patch_embs, bo_mat, be_mat, bd_mat, lam_vec)
    return (mixed, jnp.asarray(lam, dtype=jnp.float32), perm)
